# Initial kernel scaffold; baseline (speedup 1.0000x reference)
import jax
import jax.numpy as jnp
from jax.experimental import pallas as pl


def kernel(feats, edge_index, W1_0, b1_0, g1_0, be1_0, W2_0, b2_0, W1_1, b1_1, g1_1, be1_1, W2_1, b2_1):
    raise NotImplementedError("write your pallas kernel here")


# speedup vs baseline: 6.3866x; 6.3866x over previous
import functools

import jax
import jax.numpy as jnp
from jax import lax
from jax.experimental import pallas as pl
from jax.experimental.pallas import tpu as pltpu
from jax.experimental.pallas import tpu_sc as plsc

N = 10000
E = 320000
D = 128
H = 64

NC = 2
NS = 16
NW = NC * NS
EPT = E // NW
CHUNK = 80
NCHUNK = EPT // CHUNK
NPAD = 10240
ROWS_PT = NPAD // NS


@functools.cache
def _make_segsum(d):
    mesh = plsc.VectorSubcoreMesh(
        core_axis_name="c", subcore_axis_name="s", num_cores=NC, num_subcores=NS
    )

    @functools.partial(
        pl.kernel,
        out_type=jax.ShapeDtypeStruct((NC, NPAD, d), jnp.float32),
        mesh=mesh,
        scratch_types=[
            pltpu.VMEM((NCHUNK, CHUNK), jnp.int32),
            pltpu.VMEM((NCHUNK, CHUNK), jnp.int32),
            pltpu.VMEM((CHUNK, d), jnp.float32),
            pltpu.VMEM_SHARED((NPAD, d), jnp.float32),
            pltpu.SemaphoreType.DMA,
        ],
    )
    def segsum(vals_hbm, src_hbm, dst_hbm, zeros_hbm, out_hbm,
               src_v, dst_v, rows_v, acc_sh, sem):
        c = lax.axis_index("c")
        s = lax.axis_index("s")
        wid = s * NC + c
        pltpu.sync_copy(zeros_hbm, acc_sh.at[pl.ds(s * ROWS_PT, ROWS_PT)])
        pltpu.sync_copy(src_hbm.at[wid], src_v)
        pltpu.sync_copy(dst_hbm.at[wid], dst_v)
        plsc.subcore_barrier()

        @pl.loop(0, NCHUNK)
        def _(j):
            pltpu.async_copy(vals_hbm.at[src_v.at[j]], rows_v, sem).wait()
            pltpu.sync_copy(rows_v, acc_sh.at[dst_v.at[j]], add=True)

        plsc.subcore_barrier()
        pltpu.sync_copy(
            acc_sh.at[pl.ds(s * ROWS_PT, ROWS_PT)],
            out_hbm.at[c, pl.ds(s * ROWS_PT, ROWS_PT)],
        )

    return segsum


def _layer0_body(x_ref, p_ref, w1_ref, b1_ref, g1_ref, be1_ref, w2_ref, b2_ref,
                 o_ref):
    x = x_ref[...] + p_ref[0, :N] + p_ref[1, :N]
    y = jnp.dot(x, w1_ref[...], preferred_element_type=jnp.float32) + b1_ref[...]
    mu = jnp.mean(y, axis=0, keepdims=True)
    var = jnp.mean((y - mu) ** 2, axis=0, keepdims=True)
    h = g1_ref[...] * (y - mu) / jnp.sqrt(var + 1e-5) + be1_ref[...]
    h = jnp.maximum(h, 0.0)
    z = jnp.dot(h, w2_ref[...], preferred_element_type=jnp.float32) + b2_ref[...]
    o_ref[...] = jnp.pad(jnp.maximum(z, 0.0), ((0, 0), (0, D - H)))


def _layer1_body(x_ref, p_ref, w1_ref, b1_ref, g1_ref, be1_ref, w2_ref, b2_ref,
                 o_ref):
    x = x_ref[:, :H] + p_ref[0, :N, :H] + p_ref[1, :N, :H]
    y = jnp.dot(x, w1_ref[...], preferred_element_type=jnp.float32) + b1_ref[...]
    mu = jnp.mean(y, axis=0, keepdims=True)
    var = jnp.mean((y - mu) ** 2, axis=0, keepdims=True)
    h = g1_ref[...] * (y - mu) / jnp.sqrt(var + 1e-5) + be1_ref[...]
    h = jnp.maximum(h, 0.0)
    m = jnp.mean(h, axis=0, keepdims=True)
    o_ref[...] = jnp.dot(m, w2_ref[...], preferred_element_type=jnp.float32) + b2_ref[...]


def kernel(feats, edge_index, W1_0, b1_0, g1_0, be1_0, W2_0, b2_0,
           W1_1, b1_1, g1_1, be1_1, W2_1, b2_1):
    src = edge_index[0].reshape(NW, NCHUNK, CHUNK)
    dst = edge_index[1].reshape(NW, NCHUNK, CHUNK)
    zeros_d = jnp.zeros((ROWS_PT, D), jnp.float32)

    p0 = _make_segsum(D)(feats, src, dst, zeros_d)
    h = pl.pallas_call(
        _layer0_body,
        out_shape=jax.ShapeDtypeStruct((N, D), jnp.float32),
    )(feats, p0, W1_0, b1_0.reshape(1, H), g1_0.reshape(1, H),
      be1_0.reshape(1, H), W2_0, b2_0.reshape(1, H))

    p1 = _make_segsum(D)(h, src, dst, zeros_d)
    out = pl.pallas_call(
        _layer1_body,
        out_shape=jax.ShapeDtypeStruct((1, H), jnp.float32),
    )(h, p1, W1_1, b1_1.reshape(1, H), g1_1.reshape(1, H),
      be1_1.reshape(1, H), W2_1, b2_1.reshape(1, H))
    return out


# speedup vs baseline: 9.2774x; 1.4526x over previous
import functools

import jax
import jax.numpy as jnp
from jax import lax
from jax.experimental import pallas as pl
from jax.experimental.pallas import tpu as pltpu
from jax.experimental.pallas import tpu_sc as plsc

N = 10000
E = 320000
D = 128
H = 64

NC = 2
NS = 16
NW = NC * NS
EPT = E // NW
CHUNK = 100
NCHUNK = EPT // CHUNK
IBLK = 10
NBLK = NCHUNK // IBLK
NPAD = 10112
ROWS_PT = NPAD // NS


@functools.cache
def _make_segsum(d):
    mesh = plsc.VectorSubcoreMesh(
        core_axis_name="c", subcore_axis_name="s", num_cores=NC, num_subcores=NS
    )

    @functools.partial(
        pl.kernel,
        out_type=jax.ShapeDtypeStruct((NC, NPAD, d), jnp.float32),
        mesh=mesh,
        compiler_params=pltpu.CompilerParams(use_tc_tiling_on_sc=False),
        scratch_types=[
            pltpu.VMEM((IBLK, CHUNK), jnp.int32),
            pltpu.VMEM((IBLK, CHUNK), jnp.int32),
            pltpu.VMEM((IBLK, CHUNK), jnp.int32),
            pltpu.VMEM((IBLK, CHUNK), jnp.int32),
            pltpu.VMEM((CHUNK, d), jnp.float32),
            pltpu.VMEM((CHUNK, d), jnp.float32),
            pltpu.VMEM_SHARED((NPAD, d), jnp.float32),
            pltpu.SemaphoreType.DMA,
            pltpu.SemaphoreType.DMA,
        ],
    )
    def segsum(vals_hbm, src_hbm, dst_hbm, zeros_hbm, out_hbm,
               src_a, dst_a, src_b, dst_b, rows_a, rows_b, acc_sh,
               isem, gsem):
        c = lax.axis_index("c")
        s = lax.axis_index("s")
        wid = s * NC + c
        ibufs = ((src_a, dst_a), (src_b, dst_b))
        rbufs = (rows_a, rows_b)

        def load_block(k, kb):
            pltpu.async_copy(src_hbm.at[wid, k], ibufs[kb][0], isem)
            pltpu.async_copy(dst_hbm.at[wid, k], ibufs[kb][1], isem)

        def wait_block(k, kb):
            pltpu.make_async_copy(src_hbm.at[wid, k], ibufs[kb][0], isem).wait()
            pltpu.make_async_copy(dst_hbm.at[wid, k], ibufs[kb][1], isem).wait()

        load_block(0, 0)
        pltpu.sync_copy(zeros_hbm, acc_sh.at[pl.ds(s * ROWS_PT, ROWS_PT)])
        plsc.subcore_barrier()

        def run_block(k, kb):
            sa, da = ibufs[kb]
            wait_block(k, kb)

            @pl.when(k < NBLK - 1)
            def _():
                load_block(k + 1, 1 - kb)

            def start_gather(b):
                pltpu.async_copy(vals_hbm.at[sa.at[b]], rbufs[b % 2], gsem)

            def wait_gather(b):
                pltpu.make_async_copy(
                    vals_hbm.at[sa.at[b]], rbufs[b % 2], gsem).wait()

            start_gather(0)
            for b in range(IBLK):
                wait_gather(b)
                if b < IBLK - 1:
                    start_gather(b + 1)
                pltpu.sync_copy(rbufs[b % 2], acc_sh.at[da.at[b]], add=True)

        @pl.loop(0, NBLK // 2)
        def _(k2):
            for kb in range(2):
                run_block(k2 * 2 + kb, kb)

        plsc.subcore_barrier()
        pltpu.sync_copy(
            acc_sh.at[pl.ds(s * ROWS_PT, ROWS_PT)],
            out_hbm.at[c, pl.ds(s * ROWS_PT, ROWS_PT)],
        )

    return segsum


def _layer0_body(x_ref, p_ref, w1_ref, b1_ref, g1_ref, be1_ref, w2_ref, b2_ref,
                 o_ref):
    x = x_ref[...] + p_ref[0, :N] + p_ref[1, :N]
    y = jnp.dot(x, w1_ref[...], preferred_element_type=jnp.float32) + b1_ref[...]
    mu = jnp.mean(y, axis=0, keepdims=True)
    var = jnp.mean((y - mu) ** 2, axis=0, keepdims=True)
    h = g1_ref[...] * (y - mu) / jnp.sqrt(var + 1e-5) + be1_ref[...]
    h = jnp.maximum(h, 0.0)
    z = jnp.dot(h, w2_ref[...], preferred_element_type=jnp.float32) + b2_ref[...]
    o_ref[...] = jnp.maximum(z, 0.0)


def _layer1_body(x_ref, p_ref, w1_ref, b1_ref, g1_ref, be1_ref, w2_ref, b2_ref,
                 o_ref):
    x = x_ref[...] + p_ref[0, :N] + p_ref[1, :N]
    y = jnp.dot(x, w1_ref[...], preferred_element_type=jnp.float32) + b1_ref[...]
    mu = jnp.mean(y, axis=0, keepdims=True)
    var = jnp.mean((y - mu) ** 2, axis=0, keepdims=True)
    h = g1_ref[...] * (y - mu) / jnp.sqrt(var + 1e-5) + be1_ref[...]
    h = jnp.maximum(h, 0.0)
    m = jnp.mean(h, axis=0, keepdims=True)
    o_ref[...] = jnp.dot(m, w2_ref[...], preferred_element_type=jnp.float32) + b2_ref[...]


def kernel(feats, edge_index, W1_0, b1_0, g1_0, be1_0, W2_0, b2_0,
           W1_1, b1_1, g1_1, be1_1, W2_1, b2_1):
    src = edge_index[0].reshape(NW, NBLK, IBLK, CHUNK)
    dst = edge_index[1].reshape(NW, NBLK, IBLK, CHUNK)
    zeros_d = jnp.zeros((ROWS_PT, D), jnp.float32)
    zeros_h = jnp.zeros((ROWS_PT, H), jnp.float32)

    p0 = _make_segsum(D)(feats, src, dst, zeros_d)
    h = pl.pallas_call(
        _layer0_body,
        out_shape=jax.ShapeDtypeStruct((N, H), jnp.float32),
    )(feats, p0, W1_0, b1_0.reshape(1, H), g1_0.reshape(1, H),
      be1_0.reshape(1, H), W2_0, b2_0.reshape(1, H))

    p1 = _make_segsum(H)(h, src, dst, zeros_h)
    out = pl.pallas_call(
        _layer1_body,
        out_shape=jax.ShapeDtypeStruct((1, H), jnp.float32),
    )(h, p1, W1_1, b1_1.reshape(1, H), g1_1.reshape(1, H),
      be1_1.reshape(1, H), W2_1, b2_1.reshape(1, H))
    return out


# speedup vs baseline: 11.8594x; 1.2783x over previous
import functools

import jax
import jax.numpy as jnp
from jax import lax
from jax.experimental import pallas as pl
from jax.experimental.pallas import tpu as pltpu
from jax.experimental.pallas import tpu_sc as plsc

N = 10000
E = 320000
D = 128
H = 64

NC = 2
NS = 16
NW = NC * NS
EPT = E // NW
CHUNK = 50
NCHUNK = EPT // CHUNK
IBLK = 10
NBLK = NCHUNK // IBLK
NPAD = 10112
ROWS_PT = NPAD // NS


@functools.cache
def _make_segsum(d):
    mesh = plsc.VectorSubcoreMesh(
        core_axis_name="c", subcore_axis_name="s", num_cores=NC, num_subcores=NS
    )

    @functools.partial(
        pl.kernel,
        out_type=jax.ShapeDtypeStruct((NC, NPAD, d), jnp.float32),
        mesh=mesh,
        compiler_params=pltpu.CompilerParams(use_tc_tiling_on_sc=False),
        scratch_types=[
            pltpu.VMEM((IBLK, CHUNK), jnp.int32),
            pltpu.VMEM((IBLK, CHUNK), jnp.int32),
            pltpu.VMEM((IBLK, CHUNK), jnp.int32),
            pltpu.VMEM((IBLK, CHUNK), jnp.int32),
            pltpu.VMEM((CHUNK, d), jnp.float32),
            pltpu.VMEM((CHUNK, d), jnp.float32),
            pltpu.VMEM((CHUNK, d), jnp.float32),
            pltpu.VMEM((CHUNK, d), jnp.float32),
            pltpu.VMEM_SHARED((NPAD, d), jnp.float32),
            pltpu.SemaphoreType.DMA,
            pltpu.SemaphoreType.DMA((4,)),
        ],
    )
    def segsum(vals_hbm, src_hbm, dst_hbm, zeros_hbm, out_hbm,
               src_a, dst_a, src_b, dst_b, rows_a, rows_b, rows_c, rows_d,
               acc_sh, isem, gsem):
        c = lax.axis_index("c")
        s = lax.axis_index("s")
        wid = s * NC + c
        ibufs = ((src_a, dst_a), (src_b, dst_b))
        rbufs = (rows_a, rows_b, rows_c, rows_d)

        def load_block(k, kb):
            pltpu.async_copy(src_hbm.at[wid, k], ibufs[kb][0], isem)
            pltpu.async_copy(dst_hbm.at[wid, k], ibufs[kb][1], isem)

        def wait_block(k, kb):
            pltpu.make_async_copy(src_hbm.at[wid, k], ibufs[kb][0], isem).wait()
            pltpu.make_async_copy(dst_hbm.at[wid, k], ibufs[kb][1], isem).wait()

        def start_gather(sa, b, q):
            pltpu.async_copy(vals_hbm.at[sa.at[b]], rbufs[q], gsem.at[q])

        def wait_gather(sa, b, q):
            pltpu.make_async_copy(
                vals_hbm.at[sa.at[b]], rbufs[q], gsem.at[q]).wait()

        load_block(0, 0)
        pltpu.sync_copy(zeros_hbm, acc_sh.at[pl.ds(s * ROWS_PT, ROWS_PT)])
        plsc.subcore_barrier()

        def run_block(k, p, do_load, do_cross):
            sa, da = ibufs[p]
            nsa = ibufs[1 - p][0]
            if do_load:
                load_block(k + 1, 1 - p)
            for b in range(IBLK):
                q = (b + 2 * (p % 2) * (IBLK % 4 // 2)) % 4
                wait_gather(sa, b, q)
                if b < IBLK - 3:
                    start_gather(sa, b + 3, (q + 3) % 4)
                elif do_cross:
                    if b == IBLK - 3:
                        wait_block(k + 1, 1 - p)
                    start_gather(nsa, b + 3 - IBLK, (q + 3) % 4)
                pltpu.sync_copy(rbufs[q], acc_sh.at[da.at[b]], add=True)

        wait_block(0, 0)
        load_block(1, 1)
        for b in range(3):
            start_gather(ibufs[0][0], b, b)

        run_block(0, 0, False, True)

        @pl.loop(0, (NBLK - 2) // 2)
        def _(t):
            run_block(2 * t + 1, 1, True, True)
            run_block(2 * t + 2, 0, True, True)

        run_block(NBLK - 1, 1, False, False)

        plsc.subcore_barrier()
        pltpu.sync_copy(
            acc_sh.at[pl.ds(s * ROWS_PT, ROWS_PT)],
            out_hbm.at[c, pl.ds(s * ROWS_PT, ROWS_PT)],
        )

    return segsum


def _layer0_body(x_ref, p_ref, w1_ref, b1_ref, g1_ref, be1_ref, w2_ref, b2_ref,
                 o_ref):
    x = x_ref[...] + p_ref[0, :N] + p_ref[1, :N]
    y = jnp.dot(x, w1_ref[...], preferred_element_type=jnp.float32) + b1_ref[...]
    mu = jnp.mean(y, axis=0, keepdims=True)
    var = jnp.mean((y - mu) ** 2, axis=0, keepdims=True)
    h = g1_ref[...] * (y - mu) / jnp.sqrt(var + 1e-5) + be1_ref[...]
    h = jnp.maximum(h, 0.0)
    z = jnp.dot(h, w2_ref[...], preferred_element_type=jnp.float32) + b2_ref[...]
    o_ref[...] = jnp.maximum(z, 0.0)


def _layer1_body(x_ref, p_ref, w1_ref, b1_ref, g1_ref, be1_ref, w2_ref, b2_ref,
                 o_ref):
    x = x_ref[...] + p_ref[0, :N] + p_ref[1, :N]
    y = jnp.dot(x, w1_ref[...], preferred_element_type=jnp.float32) + b1_ref[...]
    mu = jnp.mean(y, axis=0, keepdims=True)
    var = jnp.mean((y - mu) ** 2, axis=0, keepdims=True)
    h = g1_ref[...] * (y - mu) / jnp.sqrt(var + 1e-5) + be1_ref[...]
    h = jnp.maximum(h, 0.0)
    m = jnp.mean(h, axis=0, keepdims=True)
    o_ref[...] = jnp.dot(m, w2_ref[...], preferred_element_type=jnp.float32) + b2_ref[...]


def kernel(feats, edge_index, W1_0, b1_0, g1_0, be1_0, W2_0, b2_0,
           W1_1, b1_1, g1_1, be1_1, W2_1, b2_1):
    src = edge_index[0].reshape(NW, NBLK, IBLK, CHUNK)
    dst = edge_index[1].reshape(NW, NBLK, IBLK, CHUNK)
    zeros_d = jnp.zeros((ROWS_PT, D), jnp.float32)
    zeros_h = jnp.zeros((ROWS_PT, H), jnp.float32)

    p0 = _make_segsum(D)(feats, src, dst, zeros_d)
    h = pl.pallas_call(
        _layer0_body,
        out_shape=jax.ShapeDtypeStruct((N, H), jnp.float32),
    )(feats, p0, W1_0, b1_0.reshape(1, H), g1_0.reshape(1, H),
      be1_0.reshape(1, H), W2_0, b2_0.reshape(1, H))

    p1 = _make_segsum(H)(h, src, dst, zeros_h)
    out = pl.pallas_call(
        _layer1_body,
        out_shape=jax.ShapeDtypeStruct((1, H), jnp.float32),
    )(h, p1, W1_1, b1_1.reshape(1, H), g1_1.reshape(1, H),
      be1_1.reshape(1, H), W2_1, b2_1.reshape(1, H))
    return out


# speedup vs baseline: 12.4507x; 1.0499x over previous
import functools

import jax
import jax.numpy as jnp
from jax import lax
from jax.experimental import pallas as pl
from jax.experimental.pallas import tpu as pltpu
from jax.experimental.pallas import tpu_sc as plsc

N = 10000
E = 320000
D = 128
H = 64

NC = 2
NS = 16
NW = NC * NS
EPT = E // NW
CHUNK = 50
NCHUNK = EPT // CHUNK
IBLK = 10
NBLK = NCHUNK // IBLK
NPAD = 10112
ROWS_PT = NPAD // NS


@functools.cache
def _make_segsum(d):
    mesh = plsc.VectorSubcoreMesh(
        core_axis_name="c", subcore_axis_name="s", num_cores=NC, num_subcores=NS
    )

    @functools.partial(
        pl.kernel,
        out_type=jax.ShapeDtypeStruct((NC, NPAD, d), jnp.float32),
        mesh=mesh,
        compiler_params=pltpu.CompilerParams(use_tc_tiling_on_sc=False),
        scratch_types=[
            pltpu.VMEM((IBLK, CHUNK), jnp.int32),
            pltpu.VMEM((IBLK, CHUNK), jnp.int32),
            pltpu.VMEM((IBLK, CHUNK), jnp.int32),
            pltpu.VMEM((IBLK, CHUNK), jnp.int32),
            pltpu.VMEM((CHUNK, d), jnp.float32),
            pltpu.VMEM((CHUNK, d), jnp.float32),
            pltpu.VMEM((CHUNK, d), jnp.float32),
            pltpu.VMEM((CHUNK, d), jnp.float32),
            pltpu.VMEM((CHUNK, d), jnp.float32),
            pltpu.VMEM_SHARED((NPAD, d), jnp.float32),
            pltpu.SemaphoreType.DMA,
            pltpu.SemaphoreType.DMA((5,)),
        ],
    )
    def segsum(vals_hbm, src_hbm, dst_hbm, zeros_hbm, out_hbm,
               src_a, dst_a, src_b, dst_b, rows_a, rows_b, rows_c, rows_d,
               rows_e, acc_sh, isem, gsem):
        c = lax.axis_index("c")
        s = lax.axis_index("s")
        wid = s * NC + c
        ibufs = ((src_a, dst_a), (src_b, dst_b))
        rbufs = (rows_a, rows_b, rows_c, rows_d, rows_e)

        def load_block(k, kb):
            pltpu.async_copy(src_hbm.at[wid, k], ibufs[kb][0], isem)
            pltpu.async_copy(dst_hbm.at[wid, k], ibufs[kb][1], isem)

        def wait_block(k, kb):
            pltpu.make_async_copy(src_hbm.at[wid, k], ibufs[kb][0], isem).wait()
            pltpu.make_async_copy(dst_hbm.at[wid, k], ibufs[kb][1], isem).wait()

        def start_gather(sa, b, q):
            pltpu.async_copy(vals_hbm.at[sa.at[b]], rbufs[q], gsem.at[q])

        def wait_gather(sa, b, q):
            pltpu.make_async_copy(
                vals_hbm.at[sa.at[b]], rbufs[q], gsem.at[q]).wait()

        load_block(0, 0)
        pltpu.sync_copy(zeros_hbm, acc_sh.at[pl.ds(s * ROWS_PT, ROWS_PT)])
        plsc.subcore_barrier()

        def run_block(k, p, do_load, do_cross):
            sa, da = ibufs[p]
            nsa = ibufs[1 - p][0]
            if do_load:
                load_block(k + 1, 1 - p)
            for b in range(IBLK):
                q = b % 5
                wait_gather(sa, b, q)
                if b < IBLK - 4:
                    start_gather(sa, b + 4, (q + 4) % 5)
                elif do_cross:
                    if b == IBLK - 4:
                        wait_block(k + 1, 1 - p)
                    start_gather(nsa, b + 4 - IBLK, (q + 4) % 5)
                pltpu.sync_copy(rbufs[q], acc_sh.at[da.at[b]], add=True)

        wait_block(0, 0)
        load_block(1, 1)
        for b in range(4):
            start_gather(ibufs[0][0], b, b)

        run_block(0, 0, False, True)

        @pl.loop(0, (NBLK - 2) // 2)
        def _(t):
            run_block(2 * t + 1, 1, True, True)
            run_block(2 * t + 2, 0, True, True)

        run_block(NBLK - 1, 1, False, False)

        plsc.subcore_barrier()
        pltpu.sync_copy(
            acc_sh.at[pl.ds(s * ROWS_PT, ROWS_PT)],
            out_hbm.at[c, pl.ds(s * ROWS_PT, ROWS_PT)],
        )

    return segsum


def _layer0_body(x_ref, p_ref, w1_ref, b1_ref, g1_ref, be1_ref, w2_ref, b2_ref,
                 o_ref):
    x = x_ref[...] + p_ref[0, :N] + p_ref[1, :N]
    y = jnp.dot(x, w1_ref[...], preferred_element_type=jnp.float32) + b1_ref[...]
    mu = jnp.mean(y, axis=0, keepdims=True)
    var = jnp.mean((y - mu) ** 2, axis=0, keepdims=True)
    h = g1_ref[...] * (y - mu) / jnp.sqrt(var + 1e-5) + be1_ref[...]
    h = jnp.maximum(h, 0.0)
    z = jnp.dot(h, w2_ref[...], preferred_element_type=jnp.float32) + b2_ref[...]
    o_ref[...] = jnp.maximum(z, 0.0)


def _layer1_body(x_ref, p_ref, w1_ref, b1_ref, g1_ref, be1_ref, w2_ref, b2_ref,
                 o_ref):
    x = x_ref[...] + p_ref[0, :N] + p_ref[1, :N]
    y = jnp.dot(x, w1_ref[...], preferred_element_type=jnp.float32) + b1_ref[...]
    mu = jnp.mean(y, axis=0, keepdims=True)
    var = jnp.mean((y - mu) ** 2, axis=0, keepdims=True)
    h = g1_ref[...] * (y - mu) / jnp.sqrt(var + 1e-5) + be1_ref[...]
    h = jnp.maximum(h, 0.0)
    m = jnp.mean(h, axis=0, keepdims=True)
    o_ref[...] = jnp.dot(m, w2_ref[...], preferred_element_type=jnp.float32) + b2_ref[...]


def kernel(feats, edge_index, W1_0, b1_0, g1_0, be1_0, W2_0, b2_0,
           W1_1, b1_1, g1_1, be1_1, W2_1, b2_1):
    src = edge_index[0].reshape(NW, NBLK, IBLK, CHUNK)
    dst = edge_index[1].reshape(NW, NBLK, IBLK, CHUNK)
    zeros_d = jnp.zeros((ROWS_PT, D), jnp.float32)
    zeros_h = jnp.zeros((ROWS_PT, H), jnp.float32)

    p0 = _make_segsum(D)(feats, src, dst, zeros_d)
    h = pl.pallas_call(
        _layer0_body,
        out_shape=jax.ShapeDtypeStruct((N, H), jnp.float32),
    )(feats, p0, W1_0, b1_0.reshape(1, H), g1_0.reshape(1, H),
      be1_0.reshape(1, H), W2_0, b2_0.reshape(1, H))

    p1 = _make_segsum(H)(h, src, dst, zeros_h)
    out = pl.pallas_call(
        _layer1_body,
        out_shape=jax.ShapeDtypeStruct((1, H), jnp.float32),
    )(h, p1, W1_1, b1_1.reshape(1, H), g1_1.reshape(1, H),
      be1_1.reshape(1, H), W2_1, b2_1.reshape(1, H))
    return out
